# baseline (speedup 1.0000x reference)
import functools

import jax
import jax.numpy as jnp
from jax import lax
from jax.experimental import pallas as pl
from jax.experimental.pallas import tpu as pltpu
from jax.experimental.pallas import tpu_sc as plsc

_N = 10000
_E = 320000
_D = 128
_H = 128
_G = 64
_AUGW = 144
_CHUNK = 128
_NCHUNKS = _E // _CHUNK
_NW = 32
_NPAD = 10240
_ROWS_PER_TILE = _NPAD // 16
_BLK = 1000
_NBLK = _N // _BLK


_EPAD = 327680
_NCH_W = 80


def _sc_edge_agg_body(yl_hbm, src_hbm, dst_hbm, zeros_hbm, out_hbm,
                      src_a, dst_a, src_b, dst_b, src_a2, dst_a2,
                      buf_a, buf_b, agg_sh, sem_a, sem_b, sem_i):
    c = lax.axis_index("c")
    s = lax.axis_index("s")
    wid = s * 2 + c
    pltpu.sync_copy(zeros_hbm,
                    agg_sh.at[pl.ds(s * _ROWS_PER_TILE, _ROWS_PER_TILE)])
    plsc.subcore_barrier()

    ebase = wid * (_NCH_W * _CHUNK)

    def start_idx(j, sref, dref):
        pltpu.async_copy(src_hbm.at[pl.ds(ebase + j * _CHUNK, _CHUNK)],
                         sref, sem_i)
        pltpu.async_copy(dst_hbm.at[pl.ds(ebase + j * _CHUNK, _CHUNK)],
                         dref, sem_i)

    def wait_idx(sref, dref):
        pltpu.make_async_copy(src_hbm.at[pl.ds(ebase, _CHUNK)],
                              sref, sem_i).wait()
        pltpu.make_async_copy(dst_hbm.at[pl.ds(ebase, _CHUNK)],
                              dref, sem_i).wait()

    def start_gather(sref, buf, sem):
        pltpu.async_copy(yl_hbm.at[sref], buf, sem)

    def wait_gather(sref, buf, sem):
        pltpu.make_async_copy(yl_hbm.at[sref], buf, sem).wait()

    start_idx(0, src_a, dst_a)
    wait_idx(src_a, dst_a)
    start_gather(src_a, buf_a, sem_a)
    start_idx(1, src_b, dst_b)

    def body(t, carry):
        j0 = 2 * t
        j1 = j0 + 1
        wait_idx(src_b, dst_b)
        start_gather(src_b, buf_b, sem_b)
        wait_gather(src_a, buf_a, sem_a)

        @pl.when(j1 + 1 < _NCH_W)
        def _():
            start_idx(j1 + 1, src_a2, dst_a2)

        pltpu.sync_copy(buf_a, agg_sh.at[dst_a], add=True)

        @pl.when(j1 + 1 < _NCH_W)
        def _():
            wait_idx(src_a2, dst_a2)
            for k in range(_CHUNK // 16):
                src_a[pl.ds(k * 16, 16)] = src_a2[pl.ds(k * 16, 16)]
                dst_a[pl.ds(k * 16, 16)] = dst_a2[pl.ds(k * 16, 16)]
            start_gather(src_a, buf_a, sem_a)
            start_idx(j1 + 2, src_b, dst_b)

        wait_gather(src_b, buf_b, sem_b)
        pltpu.sync_copy(buf_b, agg_sh.at[dst_b], add=True)
        return carry

    lax.fori_loop(0, _NCH_W // 2, body, 0)
    plsc.subcore_barrier()
    pltpu.sync_copy(agg_sh.at[pl.ds(s * _ROWS_PER_TILE, _ROWS_PER_TILE)],
                    out_hbm.at[c, pl.ds(s * _ROWS_PER_TILE, _ROWS_PER_TILE)])


def _edge_agg(yl_aug, src2d, dst2d, zeros):
    call = pl.kernel(
        _sc_edge_agg_body,
        out_type=jax.ShapeDtypeStruct((2, _NPAD, _AUGW), jnp.float32),
        mesh=plsc.VectorSubcoreMesh(core_axis_name="c", subcore_axis_name="s"),
        scratch_types=[
            pltpu.VMEM((_CHUNK,), jnp.int32),
            pltpu.VMEM((_CHUNK,), jnp.int32),
            pltpu.VMEM((_CHUNK,), jnp.int32),
            pltpu.VMEM((_CHUNK,), jnp.int32),
            pltpu.VMEM((_CHUNK,), jnp.int32),
            pltpu.VMEM((_CHUNK,), jnp.int32),
            pltpu.VMEM((_CHUNK, _AUGW), jnp.float32),
            pltpu.VMEM((_CHUNK, _AUGW), jnp.float32),
            pltpu.VMEM_SHARED((_NPAD, _AUGW), jnp.float32),
            pltpu.SemaphoreType.DMA,
            pltpu.SemaphoreType.DMA,
            pltpu.SemaphoreType.DMA,
        ],
        compiler_params=pltpu.CompilerParams(use_tc_tiling_on_sc=False),
    )
    return call(yl_aug, src2d, dst2d, zeros)


def _k1_body(nb, eb, wla, wr, w1, w2, bcol, blr, b1r, b2r,
             yl_out, base_out, mlp_out):
    xb = jnp.concatenate([nb[...], eb[...]], axis=1)
    yl_out[...] = jnp.dot(xb, wla[...], preferred_element_type=jnp.float32) + bcol[...]
    base_out[...] = jnp.dot(xb, wr[...], preferred_element_type=jnp.float32) + blr[...]
    h1 = jnp.maximum(jnp.dot(xb, w1[...], preferred_element_type=jnp.float32) + b1r[...], 0.0)
    mlp_out[...] = jnp.maximum(jnp.dot(h1, w2[...], preferred_element_type=jnp.float32) + b2r[...], 0.0)


def _k1(nf, ef, wla, wr, w1, w2, bcol, blr, b1r, b2r):
    return pl.pallas_call(
        _k1_body,
        grid=(_NBLK,),
        in_specs=[
            pl.BlockSpec((_BLK, _D), lambda i: (i, 0)),
            pl.BlockSpec((_BLK, _D), lambda i: (i, 0)),
            pl.BlockSpec((2 * _D, _AUGW), lambda i: (0, 0)),
            pl.BlockSpec((2 * _D, _H), lambda i: (0, 0)),
            pl.BlockSpec((2 * _D, _H), lambda i: (0, 0)),
            pl.BlockSpec((_H, _H), lambda i: (0, 0)),
            pl.BlockSpec((1, _AUGW), lambda i: (0, 0)),
            pl.BlockSpec((1, _H), lambda i: (0, 0)),
            pl.BlockSpec((1, _H), lambda i: (0, 0)),
            pl.BlockSpec((1, _H), lambda i: (0, 0)),
        ],
        out_specs=[
            pl.BlockSpec((_BLK, _AUGW), lambda i: (i, 0)),
            pl.BlockSpec((_BLK, _H), lambda i: (i, 0)),
            pl.BlockSpec((_BLK, _H), lambda i: (i, 0)),
        ],
        out_shape=[
            jax.ShapeDtypeStruct((_N, _AUGW), jnp.float32),
            jax.ShapeDtypeStruct((_N, _H), jnp.float32),
            jax.ShapeDtypeStruct((_N, _H), jnp.float32),
        ],
    )(nf, ef, wla, wr, w1, w2, bcol, blr, b1r, b2r)


def _k3_body(a0, a1, baseb, pre_out, sums, sumsq):
    i = pl.program_id(0)
    aggb = a0[...] + a1[...]
    deg = jnp.maximum(aggb[:, _H:_H + 1], 1.0)
    pre = aggb[:, :_H] / deg + baseb[...]
    pre_out[...] = pre

    @pl.when(i == 0)
    def _():
        sums[...] = jnp.zeros_like(sums)
        sumsq[...] = jnp.zeros_like(sumsq)

    sums[...] += jnp.sum(pre, axis=0, keepdims=True)
    sumsq[...] += jnp.sum(pre * pre, axis=0, keepdims=True)


def _k3(a0, a1, base):
    return pl.pallas_call(
        _k3_body,
        grid=(_NBLK,),
        in_specs=[
            pl.BlockSpec((_BLK, _AUGW), lambda i: (i, 0)),
            pl.BlockSpec((_BLK, _AUGW), lambda i: (i, 0)),
            pl.BlockSpec((_BLK, _H), lambda i: (i, 0)),
        ],
        out_specs=[
            pl.BlockSpec((_BLK, _H), lambda i: (i, 0)),
            pl.BlockSpec((1, _H), lambda i: (0, 0)),
            pl.BlockSpec((1, _H), lambda i: (0, 0)),
        ],
        out_shape=[
            jax.ShapeDtypeStruct((_N, _H), jnp.float32),
            jax.ShapeDtypeStruct((1, _H), jnp.float32),
            jax.ShapeDtypeStruct((1, _H), jnp.float32),
        ],
    )(a0, a1, base)


def _k4_body(preb, mlpb, batchb, sums, sumsq, gam, bet, wf, bfr,
             out, gacc, cacc):
    i = pl.program_id(0)
    mu = sums[...] / _N
    var = sumsq[...] / _N - mu * mu
    rstd = lax.rsqrt(var + 1e-5)
    xg = (preb[...] - mu) * rstd * gam[...] + bet[...]
    comb = jnp.maximum(xg, 0.0) + mlpb[...]
    b = batchb[0]
    gi = lax.broadcasted_iota(jnp.int32, (_G, 1), 0)
    oh = (gi == b).astype(jnp.float32)

    @pl.when(i == 0)
    def _():
        gacc[...] = jnp.zeros_like(gacc)
        cacc[...] = jnp.zeros_like(cacc)

    gacc[...] += jnp.dot(oh, comb, preferred_element_type=jnp.float32)
    cacc[...] += jnp.sum(oh, axis=1, keepdims=True)

    @pl.when(i == pl.num_programs(0) - 1)
    def _():
        gf = gacc[...] / jnp.maximum(cacc[...], 1.0)
        z = jnp.dot(gf, wf[...], preferred_element_type=jnp.float32) + bfr[...]
        out[...] = jax.nn.sigmoid(z)


def _k4(pre, mlp, batch3, sums, sumsq, gam, bet, wf, bfr):
    return pl.pallas_call(
        _k4_body,
        grid=(_NBLK,),
        in_specs=[
            pl.BlockSpec((_BLK, _H), lambda i: (i, 0)),
            pl.BlockSpec((_BLK, _H), lambda i: (i, 0)),
            pl.BlockSpec((1, 1, _BLK), lambda i: (i, 0, 0)),
            pl.BlockSpec((1, _H), lambda i: (0, 0)),
            pl.BlockSpec((1, _H), lambda i: (0, 0)),
            pl.BlockSpec((1, _H), lambda i: (0, 0)),
            pl.BlockSpec((1, _H), lambda i: (0, 0)),
            pl.BlockSpec((_H, 1), lambda i: (0, 0)),
            pl.BlockSpec((1, 1), lambda i: (0, 0)),
        ],
        out_specs=pl.BlockSpec((_G, 1), lambda i: (0, 0)),
        out_shape=jax.ShapeDtypeStruct((_G, 1), jnp.float32),
        scratch_shapes=[
            pltpu.VMEM((_G, _H), jnp.float32),
            pltpu.VMEM((_G, 1), jnp.float32),
        ],
    )(pre, mlp, batch3, sums, sumsq, gam, bet, wf, bfr)


def kernel(normal_features, extreme_features, Wl, bl, Wr, gamma, beta,
           W1, b1, W2, b2, Wf, bf, edge_index, batch):
    f32 = jnp.float32
    wla = jnp.concatenate([Wl, jnp.zeros((2 * _D, _AUGW - _H), f32)], axis=1)
    bcol = jnp.zeros((1, _AUGW), f32).at[0, _H].set(1.0)
    blr = bl.reshape(1, _H)
    b1r = b1.reshape(1, _H)
    b2r = b2.reshape(1, _H)
    bfr = bf.reshape(1, 1)
    gam = gamma.reshape(1, _H)
    bet = beta.reshape(1, _H)

    yl_aug, base, mlp = _k1(normal_features, extreme_features,
                            wla, Wr, W1, W2, bcol, blr, b1r, b2r)

    zeros = jnp.zeros((_ROWS_PER_TILE, _AUGW), f32)
    npad_e = _EPAD - _E
    src1 = jnp.concatenate([edge_index[0], jnp.zeros((npad_e,), jnp.int32)])
    dst1 = jnp.concatenate(
        [edge_index[1], jnp.full((npad_e,), _NPAD - 1, jnp.int32)])
    agg2 = _edge_agg(yl_aug, src1, dst1, zeros)

    pre, sums, sumsq = _k3(agg2[0, :_N], agg2[1, :_N], base)

    batch3 = batch.reshape(_NBLK, 1, _BLK)
    return _k4(pre, mlp, batch3, sums, sumsq, gam, bet, Wf, bfr)
